# baseline (speedup 1.0000x reference)
import functools

import jax
import jax.numpy as jnp
from jax import lax
from jax.experimental import pallas as pl
from jax.experimental.pallas import tpu as pltpu
from jax.experimental.pallas import tpu_sc as plsc

_N = 10000
_E = 320000
_DIN = 128
_GL = 1024
_NG = 10
_NPG = 1000
_EPS = 1e-5

_NC = 2
_NS = 16
_CHUNK = 80
_EPT = _E // _NS
_NITER = _EPT // _CHUNK
_RPT = 624
_TAIL0 = _RPT * _NS
_DRN = 16
_NDR = _RPT // _DRN

@functools.lru_cache(maxsize=None)
def _get_mesh():
  return plsc.VectorSubcoreMesh(core_axis_name="c", subcore_axis_name="s")


@functools.lru_cache(maxsize=None)
def _make_sc_scatter(w, split_cols):
  kb = 5
  nblk = 50 if split_cols else 25

  @functools.partial(
      pl.kernel,
      out_type=jax.ShapeDtypeStruct((2 * _N, w), jnp.float32),
      mesh=_get_mesh(),
      scratch_types=[
          pltpu.VMEM((2, kb, _CHUNK), jnp.int32),
          pltpu.VMEM((2, kb, _CHUNK), jnp.int32),
          pltpu.VMEM((4, _CHUNK, w), jnp.float32),
          pltpu.VMEM((_DRN, w), jnp.float32),
          pltpu.VMEM_SHARED((_N, w), jnp.float32),
          pltpu.SemaphoreType.DMA,
          pltpu.SemaphoreType.DMA,
          pltpu.SemaphoreType.DMA,
          pltpu.SemaphoreType.DMA,
          pltpu.SemaphoreType.DMA,
          pltpu.SemaphoreType.DMA,
          pltpu.SemaphoreType.DMA,
          pltpu.SemaphoreType.DMA,
      ],
  )
  def k(h_hbm, gidx_hbm, didx_hbm, out_hbm, gbuf, dbuf, rows, bounce, acc,
        gs0, gs1, gs2, gs3, ss0, ss1, ss2, ss3):
    c = lax.axis_index("c")
    s = lax.axis_index("s")
    row0 = s * _RPT
    zero16 = jnp.zeros((16,), jnp.float32)

    def zrow(i, carry):
      for j in range(w // 16):
        bounce[i, pl.ds(j * 16, 16)] = zero16
      return carry

    lax.fori_loop(0, _DRN, zrow, 0)
    for d in range(_NDR):
      pltpu.sync_copy(bounce, acc.at[pl.ds(row0 + d * _DRN, _DRN)])

    @pl.when(s == _NS - 1)
    def _():
      pltpu.sync_copy(bounce.at[pl.ds(0, 16)], acc.at[pl.ds(_TAIL0, 16)])

    plsc.subcore_barrier()

    coff = c * _N
    gsems = (gs0, gs1, gs2, gs3)
    ssems = (ss0, ss1, ss2, ss3)

    def burst(pairs):
      n = len(pairs)
      gd = [None] * n
      sd = [None] * n

      def gather(u):
        pu = u % 4
        hh, j = pairs[u]
        gd[u] = pltpu.async_copy(
            h_hbm.at[gbuf.at[hh, j]], rows.at[pu], gsems[pu]
        )

      gather(0)
      if n > 1:
        gather(1)
      for t in range(n):
        pt = t % 4
        hh, j = pairs[t]
        gd[t].wait()
        sd[t] = pltpu.async_copy(
            rows.at[pt], acc.at[dbuf.at[hh, j]], ssems[pt], add=True
        )
        u = t + 2
        if u < n:
          if u >= 4:
            sd[u - 4].wait()
          gather(u)
      for t in range(max(0, n - 4), n):
        sd[t].wait()

    def load_idx(hh, blk):
      if split_cols:
        pltpu.sync_copy(gidx_hbm.at[c, s, blk], gbuf.at[hh])
        pltpu.sync_copy(didx_hbm.at[s, blk], dbuf.at[hh])
      else:
        pltpu.sync_copy(gidx_hbm.at[c, s, blk], gbuf.at[hh])
        pltpu.sync_copy(didx_hbm.at[c, s, blk], dbuf.at[hh])

    two_kb = [(hh, j) for hh in range(2) for j in range(kb)]

    def blk_body(b, carry):
      load_idx(0, 2 * b)
      load_idx(1, 2 * b + 1)
      burst(two_kb)
      return carry

    lax.fori_loop(0, nblk // 2, blk_body, 0)
    if nblk % 2:
      load_idx(0, nblk - 1)
      burst([(0, j) for j in range(kb)])
    plsc.subcore_barrier()

    for d in range(_NDR):
      pltpu.sync_copy(acc.at[pl.ds(row0 + d * _DRN, _DRN)], bounce)
      pltpu.sync_copy(bounce, out_hbm.at[pl.ds(coff + row0 + d * _DRN, _DRN)])

    @pl.when(s == _NS - 1)
    def _():
      pltpu.sync_copy(acc.at[pl.ds(_TAIL0, 16)], bounce.at[pl.ds(0, 16)])
      pltpu.sync_copy(
          bounce.at[pl.ds(0, 16)], out_hbm.at[pl.ds(coff + _TAIL0, 16)]
      )

  return k


@functools.lru_cache(maxsize=None)
def _make_sc_degrees():
  kb = 10
  nblk = 25

  @functools.partial(
      pl.kernel,
      out_type=jax.ShapeDtypeStruct((_NC * _NS * _N,), jnp.float32),
      mesh=_get_mesh(),
      compiler_params=pltpu.CompilerParams(needs_layout_passes=False),
      scratch_types=[
          pltpu.VMEM((kb, _CHUNK), jnp.int32),
          pltpu.VMEM((_N,), jnp.float32),
      ],
  )
  def _sc_degrees(edges_hbm, out_hbm, idxb, acc):
    c = lax.axis_index("c")
    s = lax.axis_index("s")
    zero16 = jnp.zeros((16,), jnp.float32)
    one16 = jnp.ones((16,), jnp.float32)

    def zrow(i, carry):
      acc[pl.ds(i * 16, 16)] = zero16
      return carry

    lax.fori_loop(0, _N // 16, zrow, 0)

    def body(blk, carry):
      pltpu.sync_copy(edges_hbm.at[c, s, blk], idxb)
      for j in range(kb):
        for q in range(_CHUNK // 16):
          idx16 = idxb[j, pl.ds(q * 16, 16)]
          plsc.addupdate_scatter(acc, [idx16], one16)
      return carry

    lax.fori_loop(0, nblk, body, 0)

    wid = c * _NS + s
    pltpu.sync_copy(acc, out_hbm.at[pl.ds(wid * _N, _N)])

  return _sc_degrees


def _prep_body(deg_ref, f_ref, norms_ref, h0_ref):
  deg_out = jnp.sum(deg_ref[0], axis=0)
  deg_in = jnp.sum(deg_ref[1], axis=0)
  norm_out = jnp.where(deg_out > 0, lax.rsqrt(jnp.maximum(deg_out, 1.0)), 0.0)
  norm_in = jnp.where(deg_in > 0, lax.rsqrt(jnp.maximum(deg_in, 1.0)), 0.0)
  norms_ref[0, :, 0] = norm_out
  norms_ref[1, :, 0] = norm_in
  h0_ref[...] = f_ref[...] * norm_out[:, None]


def _prep(deg3, features):
  return pl.pallas_call(
      _prep_body,
      out_shape=(
          jax.ShapeDtypeStruct((2, _N, 1), jnp.float32),
          jax.ShapeDtypeStruct((_N, _DIN), jnp.float32),
      ),
  )(deg3.reshape(_NC, _NS, _N), features)


def _layer_body(d_out, do_out_scale, sum_parts, p_ref, norms_ref, w_ref, b_ref,
                out_ref):
  if sum_parts:
    x = p_ref[0] + p_ref[1]
  else:
    x = jnp.concatenate([p_ref[0], p_ref[1]], axis=-1)
  x = x * norms_ref[1, :, 0][:, None]
  y = jnp.dot(x, w_ref[...], preferred_element_type=jnp.float32) + b_ref[...]
  y = jnp.where(y > 0, y, 0.01 * y)
  mu = jnp.mean(y, axis=-1, keepdims=True)
  var = jnp.mean((y - mu) ** 2, axis=-1, keepdims=True)
  y = (y - mu) * lax.rsqrt(var + _EPS)
  if do_out_scale:
    y = y * norms_ref[0, :, 0][:, None]
  half = d_out // 2
  out_ref[0] = y[:, :half]
  out_ref[1] = y[:, half:]


def _layer_tc(p2, norms, w, b, sum_parts=False):
  d_in, d_out = w.shape
  w_in = d_in if sum_parts else d_in // 2
  r = _NPG
  return pl.pallas_call(
      functools.partial(_layer_body, d_out, True, sum_parts),
      grid=(_N // r,),
      in_specs=[
          pl.BlockSpec((2, r, w_in), lambda i: (0, i, 0)),
          pl.BlockSpec((2, r, 1), lambda i: (0, i, 0)),
          pl.BlockSpec((d_in, d_out), lambda i: (0, 0)),
          pl.BlockSpec((1, d_out), lambda i: (0, 0)),
      ],
      out_specs=pl.BlockSpec((2, r, d_out // 2), lambda i: (0, i, 0)),
      out_shape=jax.ShapeDtypeStruct((2, _N, d_out // 2), jnp.float32),
  )(p2, norms, w, b.reshape(1, d_out))


def _pool_body(p_ref, norms_ref, w_ref, b_ref, out_ref):
  x = jnp.concatenate([p_ref[0], p_ref[1]], axis=-1)
  x = x * norms_ref[1, :, 0][:, None]
  y = jnp.dot(x, w_ref[...], preferred_element_type=jnp.float32) + b_ref[...]
  y = jnp.where(y > 0, y, 0.01 * y)
  out_ref[0] = jnp.mean(y, axis=0, keepdims=True)


def _pool_tc(p2, norms, w, b):
  d_in, d_out = w.shape
  w_in = d_in // 2
  r = _NPG
  return pl.pallas_call(
      _pool_body,
      grid=(_NG,),
      in_specs=[
          pl.BlockSpec((2, r, w_in), lambda i: (0, i, 0)),
          pl.BlockSpec((2, r, 1), lambda i: (0, i, 0)),
          pl.BlockSpec((d_in, d_out), lambda i: (0, 0)),
          pl.BlockSpec((1, d_out), lambda i: (0, 0)),
      ],
      out_specs=pl.BlockSpec((1, 1, d_out), lambda i: (i, 0, 0)),
      out_shape=jax.ShapeDtypeStruct((_NG, 1, d_out), jnp.float32),
  )(p2, norms, w, b.reshape(1, d_out)).reshape(_NG, d_out)


def _head_body(h_ref, wm_ref, bm_ref, ws_ref, bs_ref, mean_ref, ls_ref):
  h = h_ref[...]
  mu = jnp.mean(h, axis=-1, keepdims=True)
  var = jnp.mean((h - mu) ** 2, axis=-1, keepdims=True)
  h = (h - mu) * lax.rsqrt(var + _EPS)
  mean_ref[...] = (
      jnp.dot(h, wm_ref[...], preferred_element_type=jnp.float32) + bm_ref[...]
  )
  ls_ref[...] = (
      jnp.dot(h, ws_ref[...], preferred_element_type=jnp.float32) + bs_ref[...]
  )


def _head_tc(pooled, wm, bm, ws, bs):
  return pl.pallas_call(
      _head_body,
      out_shape=(
          jax.ShapeDtypeStruct((_NG, _GL), jnp.float32),
          jax.ShapeDtypeStruct((_NG, _GL), jnp.float32),
      ),
  )(pooled, wm.reshape(_GL, _GL), bm.reshape(1, _GL), ws, bs.reshape(1, _GL))


def kernel(features, edge_index, batchSize, W0, b0, W1, b1, W2, b2, W3, b3,
           Wm, bm, Ws, bs):
  del batchSize
  eflat = edge_index.reshape(2 * _E)
  src = edge_index[0]
  dst = edge_index[1]
  g_split = jnp.concatenate([src, src + _N]).reshape(2, _NS, 50, 5, _CHUNK)
  d_split = dst.reshape(_NS, 50, 5, _CHUNK)
  g_esplit = src.reshape(2, _NS, 25, 5, _CHUNK)
  d_esplit = dst.reshape(2, _NS, 25, 5, _CHUNK)

  deg3 = _make_sc_degrees()(eflat.reshape(2, _NS, 25, 10, _CHUNK))
  norms, h0 = _prep(deg3.reshape(_NC, _NS, _N), features)
  sc128e = _make_sc_scatter(128, False)
  sc128 = _make_sc_scatter(128, True)
  p0 = sc128e(h0, g_esplit, d_esplit)
  h1 = _layer_tc(p0.reshape(2, _N, _DIN), norms, W0, b0, sum_parts=True)
  p1 = sc128(h1.reshape(2 * _N, 128), g_split, d_split)
  h2 = _layer_tc(p1.reshape(2, _N, 128), norms, W1, b1)
  p2 = sc128(h2.reshape(2 * _N, 128), g_split, d_split)
  h3 = _layer_tc(p2.reshape(2, _N, 128), norms, W2, b2)
  p3 = sc128(h3.reshape(2 * _N, 128), g_split, d_split)
  pooled = _pool_tc(p3.reshape(2, _N, 128), norms, W3, b3)
  return _head_tc(pooled, Wm, bm, Ws, bs)


# speedup vs baseline: 1.2262x; 1.2262x over previous
import functools

import jax
import jax.numpy as jnp
from jax import lax
from jax.experimental import pallas as pl
from jax.experimental.pallas import tpu as pltpu
from jax.experimental.pallas import tpu_sc as plsc

_N = 10000
_E = 320000
_DIN = 128
_GL = 1024
_NG = 10
_NPG = 1000
_EPS = 1e-5

_NC = 2
_NS = 16
_CHUNK = 80
_EPT = _E // _NS
_NITER = _EPT // _CHUNK
_RPT = 624
_TAIL0 = _RPT * _NS
_DRN = 48
_NDR = _RPT // _DRN

@functools.lru_cache(maxsize=None)
def _get_mesh():
  return plsc.VectorSubcoreMesh(core_axis_name="c", subcore_axis_name="s")


@functools.lru_cache(maxsize=None)
def _make_sc_scatter(w, split_cols):
  kb = 10 if split_cols else 5
  nblk = 25

  @functools.partial(
      pl.kernel,
      out_type=jax.ShapeDtypeStruct((2 * _N, w), jnp.float32),
      mesh=_get_mesh(),
      scratch_types=[
          pltpu.VMEM((2, kb, _CHUNK), jnp.int32),
          pltpu.VMEM((2, kb, _CHUNK), jnp.int32),
          pltpu.VMEM((3, _CHUNK, w), jnp.float32),
          pltpu.VMEM((_DRN, w), jnp.float32),
          pltpu.VMEM_SHARED((_N, w), jnp.float32),
          pltpu.SemaphoreType.DMA,
          pltpu.SemaphoreType.DMA,
          pltpu.SemaphoreType.DMA,
          pltpu.SemaphoreType.DMA,
          pltpu.SemaphoreType.DMA,
          pltpu.SemaphoreType.DMA,
      ],
  )
  def k(h_hbm, gidx_hbm, didx_hbm, out_hbm, gbuf, dbuf, rows, bounce, acc,
        gs0, gs1, gs2, ss0, ss1, ss2):
    c = lax.axis_index("c")
    s = lax.axis_index("s")
    row0 = s * _RPT
    zero16 = jnp.zeros((16,), jnp.float32)

    def zrow(i, carry):
      for j in range(w // 16):
        bounce[i, pl.ds(j * 16, 16)] = zero16
      return carry

    lax.fori_loop(0, _DRN, zrow, 0)
    for d in range(_NDR):
      pltpu.sync_copy(bounce, acc.at[pl.ds(row0 + d * _DRN, _DRN)])

    @pl.when(s == _NS - 1)
    def _():
      pltpu.sync_copy(bounce.at[pl.ds(0, 16)], acc.at[pl.ds(_TAIL0, 16)])

    plsc.subcore_barrier()

    coff = c * _N
    gsems = (gs0, gs1, gs2)
    ssems = (ss0, ss1, ss2)

    def burst(pairs):
      n = len(pairs)
      gd = [None] * n
      sd = [None] * n

      def gather(u):
        pu = u % 3
        hh, j = pairs[u]
        gd[u] = pltpu.async_copy(
            h_hbm.at[gbuf.at[hh, j]], rows.at[pu], gsems[pu]
        )

      gather(0)
      if n > 1:
        gather(1)
      for t in range(n):
        pt = t % 3
        hh, j = pairs[t]
        gd[t].wait()
        sd[t] = pltpu.async_copy(
            rows.at[pt], acc.at[dbuf.at[hh, j]], ssems[pt], add=True
        )
        u = t + 2
        if u < n:
          if u >= 3:
            sd[u - 3].wait()
          gather(u)
      for t in range(max(0, n - 3), n):
        sd[t].wait()

    def load_idx(hh, blk):
      if split_cols:
        pltpu.sync_copy(gidx_hbm.at[c, s, blk], gbuf.at[hh])
        pltpu.sync_copy(didx_hbm.at[s, blk], dbuf.at[hh])
      else:
        pltpu.sync_copy(gidx_hbm.at[c, s, blk], gbuf.at[hh])
        pltpu.sync_copy(didx_hbm.at[c, s, blk], dbuf.at[hh])

    two_kb = [(hh, j) for hh in range(2) for j in range(kb)]

    def blk_body(b, carry):
      load_idx(0, 2 * b)
      load_idx(1, 2 * b + 1)
      burst(two_kb)
      return carry

    lax.fori_loop(0, nblk // 2, blk_body, 0)
    if nblk % 2:
      load_idx(0, nblk - 1)
      burst([(0, j) for j in range(kb)])
    plsc.subcore_barrier()

    for d in range(_NDR):
      pltpu.sync_copy(acc.at[pl.ds(row0 + d * _DRN, _DRN)], bounce)
      pltpu.sync_copy(bounce, out_hbm.at[pl.ds(coff + row0 + d * _DRN, _DRN)])

    @pl.when(s == _NS - 1)
    def _():
      pltpu.sync_copy(acc.at[pl.ds(_TAIL0, 16)], bounce.at[pl.ds(0, 16)])
      pltpu.sync_copy(
          bounce.at[pl.ds(0, 16)], out_hbm.at[pl.ds(coff + _TAIL0, 16)]
      )

  return k


@functools.lru_cache(maxsize=None)
def _make_sc_degrees():
  kb = 10
  nblk = 25

  @functools.partial(
      pl.kernel,
      out_type=jax.ShapeDtypeStruct((_NC * _NS * _N,), jnp.float32),
      mesh=_get_mesh(),
      compiler_params=pltpu.CompilerParams(needs_layout_passes=False),
      scratch_types=[
          pltpu.VMEM((kb, _CHUNK), jnp.int32),
          pltpu.VMEM((_N,), jnp.float32),
      ],
  )
  def _sc_degrees(edges_hbm, out_hbm, idxb, acc):
    c = lax.axis_index("c")
    s = lax.axis_index("s")
    zero16 = jnp.zeros((16,), jnp.float32)
    one16 = jnp.ones((16,), jnp.float32)

    def zrow(i, carry):
      acc[pl.ds(i * 16, 16)] = zero16
      return carry

    lax.fori_loop(0, _N // 16, zrow, 0)

    def body(blk, carry):
      pltpu.sync_copy(edges_hbm.at[c, s, blk], idxb)
      for j in range(kb):
        for q in range(_CHUNK // 16):
          idx16 = idxb[j, pl.ds(q * 16, 16)]
          plsc.addupdate_scatter(acc, [idx16], one16)
      return carry

    lax.fori_loop(0, nblk, body, 0)

    wid = c * _NS + s
    pltpu.sync_copy(acc, out_hbm.at[pl.ds(wid * _N, _N)])

  return _sc_degrees


def _prep_body(deg_ref, f_ref, norms_ref, h0_ref):
  deg_out = jnp.sum(deg_ref[0], axis=0)
  deg_in = jnp.sum(deg_ref[1], axis=0)
  norm_out = jnp.where(deg_out > 0, lax.rsqrt(jnp.maximum(deg_out, 1.0)), 0.0)
  norm_in = jnp.where(deg_in > 0, lax.rsqrt(jnp.maximum(deg_in, 1.0)), 0.0)
  norms_ref[0, :, 0] = norm_out
  norms_ref[1, :, 0] = norm_in
  h0_ref[...] = f_ref[...] * norm_out[:, None]


def _prep(deg3, features):
  return pl.pallas_call(
      _prep_body,
      out_shape=(
          jax.ShapeDtypeStruct((2, _N, 1), jnp.float32),
          jax.ShapeDtypeStruct((_N, _DIN), jnp.float32),
      ),
  )(deg3.reshape(_NC, _NS, _N), features)


def _layer_body(d_out, do_out_scale, sum_parts, p_ref, norms_ref, w_ref, b_ref,
                out_ref):
  if sum_parts:
    x = p_ref[0] + p_ref[1]
  else:
    x = jnp.concatenate([p_ref[0], p_ref[1]], axis=-1)
  x = x * norms_ref[1, :, 0][:, None]
  y = jnp.dot(x, w_ref[...], preferred_element_type=jnp.float32) + b_ref[...]
  y = jnp.where(y > 0, y, 0.01 * y)
  mu = jnp.mean(y, axis=-1, keepdims=True)
  var = jnp.mean((y - mu) ** 2, axis=-1, keepdims=True)
  y = (y - mu) * lax.rsqrt(var + _EPS)
  if do_out_scale:
    y = y * norms_ref[0, :, 0][:, None]
  half = d_out // 2
  out_ref[0] = y[:, :half]
  out_ref[1] = y[:, half:]


def _layer_tc(p2, norms, w, b, sum_parts=False):
  d_in, d_out = w.shape
  w_in = d_in if sum_parts else d_in // 2
  r = _NPG
  return pl.pallas_call(
      functools.partial(_layer_body, d_out, True, sum_parts),
      grid=(_N // r,),
      in_specs=[
          pl.BlockSpec((2, r, w_in), lambda i: (0, i, 0)),
          pl.BlockSpec((2, r, 1), lambda i: (0, i, 0)),
          pl.BlockSpec((d_in, d_out), lambda i: (0, 0)),
          pl.BlockSpec((1, d_out), lambda i: (0, 0)),
      ],
      out_specs=pl.BlockSpec((2, r, d_out // 2), lambda i: (0, i, 0)),
      out_shape=jax.ShapeDtypeStruct((2, _N, d_out // 2), jnp.float32),
  )(p2, norms, w, b.reshape(1, d_out))


def _pool_body(p_ref, norms_ref, w_ref, b_ref, out_ref):
  x = jnp.concatenate([p_ref[0], p_ref[1]], axis=-1)
  x = x * norms_ref[1, :, 0][:, None]
  y = jnp.dot(x, w_ref[...], preferred_element_type=jnp.float32) + b_ref[...]
  y = jnp.where(y > 0, y, 0.01 * y)
  out_ref[0] = jnp.mean(y, axis=0, keepdims=True)


def _pool_tc(p2, norms, w, b):
  d_in, d_out = w.shape
  w_in = d_in // 2
  r = _NPG
  return pl.pallas_call(
      _pool_body,
      grid=(_NG,),
      in_specs=[
          pl.BlockSpec((2, r, w_in), lambda i: (0, i, 0)),
          pl.BlockSpec((2, r, 1), lambda i: (0, i, 0)),
          pl.BlockSpec((d_in, d_out), lambda i: (0, 0)),
          pl.BlockSpec((1, d_out), lambda i: (0, 0)),
      ],
      out_specs=pl.BlockSpec((1, 1, d_out), lambda i: (i, 0, 0)),
      out_shape=jax.ShapeDtypeStruct((_NG, 1, d_out), jnp.float32),
  )(p2, norms, w, b.reshape(1, d_out)).reshape(_NG, d_out)


def _head_body(h_ref, wm_ref, bm_ref, ws_ref, bs_ref, mean_ref, ls_ref):
  h = h_ref[...]
  mu = jnp.mean(h, axis=-1, keepdims=True)
  var = jnp.mean((h - mu) ** 2, axis=-1, keepdims=True)
  h = (h - mu) * lax.rsqrt(var + _EPS)
  mean_ref[...] = (
      jnp.dot(h, wm_ref[...], preferred_element_type=jnp.float32) + bm_ref[...]
  )
  ls_ref[...] = (
      jnp.dot(h, ws_ref[...], preferred_element_type=jnp.float32) + bs_ref[...]
  )


def _head_tc(pooled, wm, bm, ws, bs):
  return pl.pallas_call(
      _head_body,
      out_shape=(
          jax.ShapeDtypeStruct((_NG, _GL), jnp.float32),
          jax.ShapeDtypeStruct((_NG, _GL), jnp.float32),
      ),
  )(pooled, wm.reshape(_GL, _GL), bm.reshape(1, _GL), ws, bs.reshape(1, _GL))


def kernel(features, edge_index, batchSize, W0, b0, W1, b1, W2, b2, W3, b3,
           Wm, bm, Ws, bs):
  del batchSize
  eflat = edge_index.reshape(2 * _E)
  src = edge_index[0]
  dst = edge_index[1]
  g_split = jnp.concatenate([src, src + _N]).reshape(2, _NS, 25, 10, _CHUNK)
  d_split = dst.reshape(_NS, 25, 10, _CHUNK)
  g_esplit = src.reshape(2, _NS, 25, 5, _CHUNK)
  d_esplit = dst.reshape(2, _NS, 25, 5, _CHUNK)

  deg3 = _make_sc_degrees()(eflat.reshape(2, _NS, 25, 10, _CHUNK))
  norms, h0 = _prep(deg3.reshape(_NC, _NS, _N), features)
  sc128e = _make_sc_scatter(128, False)
  sc128 = _make_sc_scatter(128, True)
  p0 = sc128e(h0, g_esplit, d_esplit)
  h1 = _layer_tc(p0.reshape(2, _N, _DIN), norms, W0, b0, sum_parts=True)
  p1 = sc128(h1.reshape(2 * _N, 128), g_split, d_split)
  h2 = _layer_tc(p1.reshape(2, _N, 128), norms, W1, b1)
  p2 = sc128(h2.reshape(2 * _N, 128), g_split, d_split)
  h3 = _layer_tc(p2.reshape(2, _N, 128), norms, W2, b2)
  p3 = sc128(h3.reshape(2 * _N, 128), g_split, d_split)
  pooled = _pool_tc(p3.reshape(2, _N, 128), norms, W3, b3)
  return _head_tc(pooled, Wm, bm, Ws, bs)
